# initial kernel scaffold (unmeasured)
import jax
import jax.numpy as jnp
from jax import lax
from jax.experimental import pallas as pl
from jax.experimental.pallas import tpu as pltpu

B, SQ, H, D = 4, 32, 8, 128
SKV = 4096
SCALE = D ** -0.5


def _local_attn_body(q_ref, k_ref, v_ref, o_ref, m_ref, l_ref):
    q = q_ref[0, :, 0, :]
    k = k_ref[0, :, 0, :]
    v = v_ref[0, :, 0, :]
    s = lax.dot_general(
        q, k, (((1,), (1,)), ((), ())), preferred_element_type=jnp.float32
    ) * SCALE
    m = jnp.max(s, axis=1, keepdims=True)
    p = jnp.exp(s - m)
    l = jnp.sum(p, axis=1, keepdims=True)
    o = lax.dot_general(
        p, v, (((1,), (0,)), ((), ())), preferred_element_type=jnp.float32
    )
    o_ref[0, :, 0, :] = o
    m_ref[...] = m.reshape(1, SQ, 1)
    l_ref[...] = l.reshape(1, SQ, 1)


def _combine_body(
    o_ref, m_ref, l_ref, out_ref, ro_ref, rm_ref, rl_ref, send_sems, recv_sems
):
    my_x = lax.axis_index("x")
    my_y = lax.axis_index("y")
    my_z = lax.axis_index("z")
    nbr = (my_x, 1 - my_y, my_z)

    barrier = pltpu.get_barrier_semaphore()
    pl.semaphore_signal(
        barrier, inc=1, device_id=nbr, device_id_type=pl.DeviceIdType.MESH
    )
    pl.semaphore_wait(barrier, 1)

    copies = []
    for i, (src, dst) in enumerate(
        ((o_ref, ro_ref), (m_ref, rm_ref), (l_ref, rl_ref))
    ):
        c = pltpu.make_async_remote_copy(
            src_ref=src,
            dst_ref=dst,
            send_sem=send_sems.at[i],
            recv_sem=recv_sems.at[i],
            device_id=nbr,
            device_id_type=pl.DeviceIdType.MESH,
        )
        c.start()
        copies.append(c)
    for c in copies:
        c.wait()

    m1 = m_ref[...]
    l1 = l_ref[...]
    m2 = rm_ref[...]
    l2 = rl_ref[...]
    mx = jnp.maximum(m1, m2)
    a1 = jnp.exp(m1 - mx)
    a2 = jnp.exp(m2 - mx)
    denom = a1 * l1 + a2 * l2
    w1 = (a1 / denom)[..., None]
    w2 = (a2 / denom)[..., None]
    out_ref[...] = w1 * o_ref[...] + w2 * ro_ref[...]


def kernel(Q, K, V):
    o_un, m, l = pl.pallas_call(
        _local_attn_body,
        grid=(B, H),
        in_specs=[
            pl.BlockSpec((1, SQ, 1, D), lambda b, h: (b, 0, h, 0)),
            pl.BlockSpec((1, SKV, 1, D), lambda b, h: (b, 0, h, 0)),
            pl.BlockSpec((1, SKV, 1, D), lambda b, h: (b, 0, h, 0)),
        ],
        out_specs=[
            pl.BlockSpec((1, SQ, 1, D), lambda b, h: (b, 0, h, 0)),
            pl.BlockSpec((1, SQ, 1), lambda b, h: (b, 0, h)),
            pl.BlockSpec((1, SQ, 1), lambda b, h: (b, 0, h)),
        ],
        out_shape=[
            jax.ShapeDtypeStruct((B, SQ, H, D), jnp.float32),
            jax.ShapeDtypeStruct((B, SQ, H), jnp.float32),
            jax.ShapeDtypeStruct((B, SQ, H), jnp.float32),
        ],
    )(Q, K, V)

    return pl.pallas_call(
        _combine_body,
        out_shape=jax.ShapeDtypeStruct((B, SQ, H, D), jnp.float32),
        in_specs=[pl.BlockSpec(memory_space=pltpu.VMEM)] * 3,
        out_specs=pl.BlockSpec(memory_space=pltpu.VMEM),
        scratch_shapes=[
            pltpu.VMEM((B, SQ, H, D), jnp.float32),
            pltpu.VMEM((B, SQ, H), jnp.float32),
            pltpu.VMEM((B, SQ, H), jnp.float32),
            pltpu.SemaphoreType.DMA((3,)),
            pltpu.SemaphoreType.DMA((3,)),
        ],
        compiler_params=pltpu.CompilerParams(collective_id=0),
    )(o_un, m, l)


# baseline (device time: 213082 ns/iter reference)
import jax
import jax.numpy as jnp
from jax import lax
from jax.experimental import pallas as pl
from jax.experimental.pallas import tpu as pltpu

B, SQ, H, D = 4, 32, 8, 128
SKV = 4096
SCALE = D ** -0.5


KC = 1024
NKC = SKV // KC


def _local_attn_body(
    q_ref, k_ref, v_ref, o_ref, m_ref, l_ref, acc_o, acc_m, acc_l
):
    kc = pl.program_id(1)

    @pl.when(kc == 0)
    def _():
        acc_m[...] = jnp.full((SQ, H), -jnp.inf, jnp.float32)
        acc_l[...] = jnp.zeros((SQ, H), jnp.float32)
        acc_o[...] = jnp.zeros((SQ, H * D), jnp.float32)

    for h in range(H):
        sl = slice(h * D, (h + 1) * D)
        q = q_ref[0, :, sl]
        k = k_ref[0, :, sl]
        v = v_ref[0, :, sl]
        s = lax.dot_general(
            q, k, (((1,), (1,)), ((), ())), preferred_element_type=jnp.float32
        ) * SCALE
        m_prev = acc_m[:, h : h + 1]
        m_cur = jnp.max(s, axis=1, keepdims=True)
        m_new = jnp.maximum(m_prev, m_cur)
        alpha = jnp.exp(m_prev - m_new)
        p = jnp.exp(s - m_new)
        l_new = acc_l[:, h : h + 1] * alpha + jnp.sum(p, axis=1, keepdims=True)
        o_new = acc_o[:, sl] * alpha + lax.dot_general(
            p, v, (((1,), (0,)), ((), ())), preferred_element_type=jnp.float32
        )
        acc_m[:, h : h + 1] = m_new
        acc_l[:, h : h + 1] = l_new
        acc_o[:, sl] = o_new

    @pl.when(kc == NKC - 1)
    def _():
        o_ref[0, :, :] = acc_o[...]
        m_ref[0, :, :] = acc_m[...]
        l_ref[0, :, :] = acc_l[...]


def _combine_body(
    o_ref, m_ref, l_ref, out_ref, ro_ref, rm_ref, rl_ref, send_sems, recv_sems
):
    my_x = lax.axis_index("x")
    my_y = lax.axis_index("y")
    my_z = lax.axis_index("z")
    nbr = (my_x, 1 - my_y, my_z)

    barrier = pltpu.get_barrier_semaphore()
    pl.semaphore_signal(
        barrier, inc=1, device_id=nbr, device_id_type=pl.DeviceIdType.MESH
    )
    pl.semaphore_wait(barrier, 1)

    copies = []
    for i, (src, dst) in enumerate(
        ((o_ref, ro_ref), (m_ref, rm_ref), (l_ref, rl_ref))
    ):
        c = pltpu.make_async_remote_copy(
            src_ref=src,
            dst_ref=dst,
            send_sem=send_sems.at[i],
            recv_sem=recv_sems.at[i],
            device_id=nbr,
            device_id_type=pl.DeviceIdType.MESH,
        )
        c.start()
        copies.append(c)
    for c in copies:
        c.wait()

    m1 = m_ref[...]
    l1 = l_ref[...]
    m2 = rm_ref[...]
    l2 = rl_ref[...]
    mx = jnp.maximum(m1, m2)
    a1 = jnp.exp(m1 - mx)
    a2 = jnp.exp(m2 - mx)
    denom = a1 * l1 + a2 * l2
    w1 = (a1 / denom)[..., None]
    w2 = (a2 / denom)[..., None]
    out_ref[...] = w1 * o_ref[...] + w2 * ro_ref[...]


def kernel(Q, K, V):
    q2 = Q.reshape(B, SQ, H * D)
    k2 = K.reshape(B, SKV, H * D)
    v2 = V.reshape(B, SKV, H * D)

    o_un, m, l = pl.pallas_call(
        _local_attn_body,
        grid=(B, NKC),
        in_specs=[
            pl.BlockSpec((1, SQ, H * D), lambda b, kc: (b, 0, 0)),
            pl.BlockSpec((1, KC, H * D), lambda b, kc: (b, kc, 0)),
            pl.BlockSpec((1, KC, H * D), lambda b, kc: (b, kc, 0)),
        ],
        out_specs=[
            pl.BlockSpec((1, SQ, H * D), lambda b, kc: (b, 0, 0)),
            pl.BlockSpec((1, SQ, H), lambda b, kc: (b, 0, 0)),
            pl.BlockSpec((1, SQ, H), lambda b, kc: (b, 0, 0)),
        ],
        out_shape=[
            jax.ShapeDtypeStruct((B, SQ, H * D), jnp.float32),
            jax.ShapeDtypeStruct((B, SQ, H), jnp.float32),
            jax.ShapeDtypeStruct((B, SQ, H), jnp.float32),
        ],
        scratch_shapes=[
            pltpu.VMEM((SQ, H * D), jnp.float32),
            pltpu.VMEM((SQ, H), jnp.float32),
            pltpu.VMEM((SQ, H), jnp.float32),
        ],
    )(q2, k2, v2)

    out = pl.pallas_call(
        _combine_body,
        out_shape=jax.ShapeDtypeStruct((B, SQ, H, D), jnp.float32),
        in_specs=[pl.BlockSpec(memory_space=pltpu.VMEM)] * 3,
        out_specs=pl.BlockSpec(memory_space=pltpu.VMEM),
        scratch_shapes=[
            pltpu.VMEM((B, SQ, H, D), jnp.float32),
            pltpu.VMEM((B, SQ, H), jnp.float32),
            pltpu.VMEM((B, SQ, H), jnp.float32),
            pltpu.SemaphoreType.DMA((3,)),
            pltpu.SemaphoreType.DMA((3,)),
        ],
        compiler_params=pltpu.CompilerParams(collective_id=0),
    )(o_un.reshape(B, SQ, H, D), m, l)
    return out
